# baseline (speedup 1.0000x reference)
import functools

import jax
import jax.numpy as jnp
from jax import lax
from jax.experimental import pallas as pl
from jax.experimental.pallas import tpu as pltpu
from jax.experimental.pallas import tpu_sc as plsc


def _tc_body(tab_ref, out_ref):
    block = tab_ref[...]
    out_ref[...] = jnp.broadcast_to(block[None, :, :], out_ref.shape)


def _tc_broadcast(table, n_batch, block_s=512):
    s, h = table.shape
    return pl.pallas_call(
        _tc_body,
        grid=(s // block_s,),
        in_specs=[pl.BlockSpec((block_s, h), lambda i: (i, 0))],
        out_specs=pl.BlockSpec((n_batch, block_s, h), lambda i: (0, i, 0)),
        out_shape=jax.ShapeDtypeStruct((n_batch, s, h), table.dtype),
    )(table)


def _make_sc_copy(s, h, dtype):
    info = plsc.get_sparse_core_info()
    nc, ns = info.num_cores, info.num_subcores
    nw = nc * ns
    rows_per_w = s // nw
    chunk = 64
    n_chunks = rows_per_w // chunk
    mesh = plsc.VectorSubcoreMesh(core_axis_name="c", subcore_axis_name="s")

    @functools.partial(
        pl.kernel,
        mesh=mesh,
        out_type=jax.ShapeDtypeStruct((s, h), dtype),
        scratch_types=[pltpu.VMEM((chunk, h), dtype)],
    )
    def sc_copy(table_hbm, out_hbm, buf):
        wid = lax.axis_index("s") * nc + lax.axis_index("c")
        base = wid * rows_per_w
        for c in range(n_chunks):
            lo = base + c * chunk
            pltpu.sync_copy(table_hbm.at[pl.ds(lo, chunk)], buf)
            pltpu.sync_copy(buf, out_hbm.at[pl.ds(lo, chunk)])

    return sc_copy


def kernel(x, pos_embedding):
    b = x.shape[0]
    s, h = pos_embedding.shape
    tc_part = _tc_broadcast(pos_embedding, b - 1)
    sc_part = _make_sc_copy(s, h, pos_embedding.dtype)(pos_embedding)
    flat = jnp.concatenate([tc_part.reshape((b - 1) * s, h), sc_part], axis=0)
    return flat.reshape(b, s, h)


# speedup vs baseline: 1.5550x; 1.5550x over previous
import functools

import jax
import jax.numpy as jnp
from jax import lax
from jax.experimental import pallas as pl
from jax.experimental.pallas import tpu as pltpu
from jax.experimental.pallas import tpu_sc as plsc


def _tc_body(tab_ref, _, out_ref):
    block = tab_ref[...]
    out_ref[...] = jnp.broadcast_to(block[None, :, :], out_ref.shape)


def _tc_fill_front(table, partial, n_written, block_s=512):
    b, s, h = partial.shape
    return pl.pallas_call(
        _tc_body,
        grid=(s // block_s, n_written),
        in_specs=[
            pl.BlockSpec((block_s, h), lambda i, j: (i, 0)),
            pl.BlockSpec(memory_space=pl.ANY),
        ],
        out_specs=pl.BlockSpec((1, block_s, h), lambda i, j: (j, i, 0)),
        out_shape=jax.ShapeDtypeStruct((b, s, h), table.dtype),
        input_output_aliases={1: 0},
    )(table, partial)


def _make_sc_batch3(b, s, h, dtype):
    info = plsc.get_sparse_core_info()
    nc, ns = info.num_cores, info.num_subcores
    nw = nc * ns
    rows_per_w = s // nw
    chunk = 64
    n_chunks = rows_per_w // chunk
    mesh = plsc.VectorSubcoreMesh(core_axis_name="c", subcore_axis_name="s")

    @functools.partial(
        pl.kernel,
        mesh=mesh,
        out_type=jax.ShapeDtypeStruct((b, s, h), dtype),
        scratch_types=[pltpu.VMEM((chunk, h), dtype)],
    )
    def sc_batch3(table_hbm, out_hbm, buf):
        wid = lax.axis_index("s") * nc + lax.axis_index("c")
        base = wid * rows_per_w
        for c in range(n_chunks):
            lo = base + c * chunk
            pltpu.sync_copy(table_hbm.at[pl.ds(lo, chunk)], buf)
            pltpu.sync_copy(buf, out_hbm.at[b - 1, pl.ds(lo, chunk)])

    return sc_batch3


def kernel(x, pos_embedding):
    b = x.shape[0]
    s, h = pos_embedding.shape
    partial = _make_sc_batch3(b, s, h, pos_embedding.dtype)(pos_embedding)
    return _tc_fill_front(pos_embedding, partial, b - 1)


# speedup vs baseline: 2.2722x; 1.4612x over previous
import functools

import jax
import jax.numpy as jnp
from jax import lax
from jax.experimental import pallas as pl
from jax.experimental.pallas import tpu as pltpu
from jax.experimental.pallas import tpu_sc as plsc


def _make_sc_broadcast(b, s, h, dtype):
    info = plsc.get_sparse_core_info()
    nc, ns = info.num_cores, info.num_subcores
    nw = nc * ns
    rows_per_w = s // nw
    chunk = 64
    n_chunks = rows_per_w // chunk
    mesh = plsc.VectorSubcoreMesh(core_axis_name="c", subcore_axis_name="s")

    @functools.partial(
        pl.kernel,
        mesh=mesh,
        out_type=jax.ShapeDtypeStruct((b, s, h), dtype),
        scratch_types=[pltpu.VMEM((chunk, h), dtype), pltpu.SemaphoreType.DMA],
    )
    def sc_broadcast(table_hbm, out_hbm, buf, wsem):
        wid = lax.axis_index("s") * nc + lax.axis_index("c")
        base = wid * rows_per_w
        for c in range(n_chunks):
            lo = base + c * chunk
            pltpu.sync_copy(table_hbm.at[pl.ds(lo, chunk)], buf)
            writes = [
                pltpu.async_copy(buf, out_hbm.at[bi, pl.ds(lo, chunk)], wsem)
                for bi in range(b)
            ]
            for w in writes:
                w.wait()

    return sc_broadcast


def kernel(x, pos_embedding):
    b = x.shape[0]
    s, h = pos_embedding.shape
    return _make_sc_broadcast(b, s, h, pos_embedding.dtype)(pos_embedding)
